# baseline (speedup 1.0000x reference)
import functools

import jax
import jax.numpy as jnp
from jax import lax
from jax.experimental import pallas as pl
from jax.experimental.pallas import tpu as pltpu
from jax.experimental.pallas import tpu_sc as plsc

SEQLEN = 65536
TOKENSIZE = 256
WINDOW = 4096

N = WINDOW * TOKENSIZE
FLAT = SEQLEN * TOKENSIZE
NC, NS = 2, 16
NW = NC * NS
CHUNK = N // NW


@functools.partial(
    pl.kernel,
    mesh=plsc.VectorSubcoreMesh(core_axis_name="c", subcore_axis_name="s"),
    out_type=jax.ShapeDtypeStruct((N,), jnp.float32),
    scratch_types=[
        pltpu.VMEM((CHUNK,), jnp.int32),
        pltpu.VMEM((CHUNK,), jnp.float32),
        pltpu.SemaphoreType.DMA,
    ],
)
def _sc_gather(table_hbm, idx_hbm, out_hbm, idx_v, val_v, sem):
    wid = lax.axis_index("s") * NC + lax.axis_index("c")
    base = wid * CHUNK
    pltpu.sync_copy(idx_hbm.at[pl.ds(base, CHUNK)], idx_v)
    pltpu.async_copy(table_hbm.at[idx_v], val_v, sem).wait()
    pltpu.sync_copy(val_v, out_hbm.at[pl.ds(base, CHUNK)])


def kernel(ddx, mdx, p, perm):
    table = ddx.reshape(FLAT)
    idx = jax.lax.slice(perm, (0, 0), (WINDOW, TOKENSIZE)).reshape(N)
    wdx = _sc_gather(table, idx)
    ddx_out = jnp.stack([ddx[:WINDOW], wdx.reshape(WINDOW, TOKENSIZE)])
    return (ddx_out, mdx[:WINDOW], p[:WINDOW])
